# baseline (speedup 1.0000x reference)
import functools

import jax
import jax.numpy as jnp
from jax import lax
from jax.experimental import pallas as pl
from jax.experimental.pallas import tpu as pltpu
from jax.experimental.pallas import tpu_sc as plsc

N = 10000
HID = 128
D4 = 512
GEN_EPS = 1e-7
RB = 1000


def _ln(t, g, b, eps=1e-5):
    mu = jnp.mean(t, axis=-1, keepdims=True)
    var = jnp.mean((t - mu) ** 2, axis=-1, keepdims=True)
    return (t - mu) / jnp.sqrt(var + eps) * g + b


def _fc_body(x_ref, w_ref, b_ref, o_ref):
    o_ref[...] = jnp.maximum(
        jnp.dot(x_ref[...], w_ref[...], preferred_element_type=jnp.float32)
        + b_ref[...], 0.0)


def _fc(x, W, b):
    return pl.pallas_call(
        _fc_body,
        grid=(N // RB,),
        in_specs=[
            pl.BlockSpec((RB, 1024), lambda i: (i, 0)),
            pl.BlockSpec((1024, HID), lambda i: (0, 0)),
            pl.BlockSpec((1, HID), lambda i: (0, 0)),
        ],
        out_specs=pl.BlockSpec((RB, HID), lambda i: (i, 0)),
        out_shape=jax.ShapeDtypeStruct((N, HID), jnp.float32),
    )(x, W, b.reshape(1, HID))


def _conv_body(res, h_ref, den_ref, num_ref, w1_ref, b1_ref, g1_ref,
               be1_ref, w2_ref, b2_ref, ng_ref, nb_ref, o_ref):
    h = h_ref[...]
    den = jnp.concatenate([den_ref[0], den_ref[1]], axis=-1)
    num = jnp.concatenate([num_ref[0], num_ref[1]], axis=-1)
    agg = num / (den + 1e-16)
    out = agg + h
    t = jnp.dot(out, w1_ref[...], preferred_element_type=jnp.float32) + b1_ref[...]
    t = jnp.maximum(_ln(t, g1_ref[...], be1_ref[...]), 0.0)
    r = jnp.dot(t, w2_ref[...], preferred_element_type=jnp.float32) + b2_ref[...]
    if res:
        r = jnp.maximum(_ln(r, ng_ref[...], nb_ref[...]), 0.0)
        o_ref[...] = h + r
    else:
        o_ref[...] = r


def _conv_dense(h, den, num, w1, b1, g1, be1, w2, b2, ng, nb, res):
    return pl.pallas_call(
        functools.partial(_conv_body, res),
        grid=(N // RB,),
        in_specs=[
            pl.BlockSpec((RB, HID), lambda i: (i, 0)),
            pl.BlockSpec((2, RB, 64), lambda i: (0, i, 0)),
            pl.BlockSpec((2, RB, 64), lambda i: (0, i, 0)),
            pl.BlockSpec((HID, 2 * HID), lambda i: (0, 0)),
            pl.BlockSpec((1, 2 * HID), lambda i: (0, 0)),
            pl.BlockSpec((1, 2 * HID), lambda i: (0, 0)),
            pl.BlockSpec((1, 2 * HID), lambda i: (0, 0)),
            pl.BlockSpec((2 * HID, HID), lambda i: (0, 0)),
            pl.BlockSpec((1, HID), lambda i: (0, 0)),
            pl.BlockSpec((1, HID), lambda i: (0, 0)),
            pl.BlockSpec((1, HID), lambda i: (0, 0)),
        ],
        out_specs=pl.BlockSpec((RB, HID), lambda i: (i, 0)),
        out_shape=jax.ShapeDtypeStruct((N, HID), jnp.float32),
    )(h, den, num, w1, b1.reshape(1, -1), g1.reshape(1, -1),
      be1.reshape(1, -1), w2, b2.reshape(1, -1), ng.reshape(1, -1),
      nb.reshape(1, -1))


def _head1_body(x_ref, wphi_ref, bphi_ref, wa_ref, ba_ref, wb_ref, bb_ref,
                wc_ref, bc_ref, hp_ref, s_ref):
    hp = jnp.maximum(
        jnp.dot(x_ref[...], wphi_ref[...], preferred_element_type=jnp.float32)
        + bphi_ref[...], 0.0)
    a = jnp.tanh(jnp.dot(hp, wa_ref[...], preferred_element_type=jnp.float32)
                 + ba_ref[...])
    b = jax.nn.sigmoid(
        jnp.dot(hp, wb_ref[...], preferred_element_type=jnp.float32)
        + bb_ref[...])
    s = jnp.dot(a * b, wc_ref[...], preferred_element_type=jnp.float32) + bc_ref[...]
    hp_ref[...] = hp
    s_ref[...] = s


def _head1(x_, p):
    return pl.pallas_call(
        _head1_body,
        grid=(N // RB,),
        in_specs=[
            pl.BlockSpec((RB, D4), lambda i: (i, 0)),
            pl.BlockSpec((D4, D4), lambda i: (0, 0)),
            pl.BlockSpec((1, D4), lambda i: (0, 0)),
            pl.BlockSpec((D4, D4), lambda i: (0, 0)),
            pl.BlockSpec((1, D4), lambda i: (0, 0)),
            pl.BlockSpec((D4, D4), lambda i: (0, 0)),
            pl.BlockSpec((1, D4), lambda i: (0, 0)),
            pl.BlockSpec((D4, 1), lambda i: (0, 0)),
            pl.BlockSpec((1, 1), lambda i: (0, 0)),
        ],
        out_specs=[
            pl.BlockSpec((RB, D4), lambda i: (i, 0)),
            pl.BlockSpec((RB, 1), lambda i: (i, 0)),
        ],
        out_shape=[
            jax.ShapeDtypeStruct((N, D4), jnp.float32),
            jax.ShapeDtypeStruct((N, 1), jnp.float32),
        ],
    )(x_, p["W_phi"], p["b_phi"].reshape(1, -1), p["Wa"], p["ba"].reshape(1, -1),
      p["Wb"], p["bb"].reshape(1, -1), p["Wc"], p["bc"].reshape(1, -1))


def _head2_body(s_ref, hp_ref, wr_ref, br_ref, wcls_ref, bcls_ref, o_ref):
    s = s_ref[...]
    m = jnp.max(s)
    e = jnp.exp(s - m)
    w = e / jnp.sum(e)
    hpool = jnp.sum(w * hp_ref[...], axis=0, keepdims=True)
    hh = jnp.maximum(
        jnp.dot(hpool, wr_ref[...], preferred_element_type=jnp.float32)
        + br_ref[...], 0.0)
    o_ref[...] = (jnp.dot(hh, wcls_ref[...], preferred_element_type=jnp.float32)
                  + bcls_ref[...])


def _head2(s, hp, p):
    return pl.pallas_call(
        _head2_body,
        out_shape=jax.ShapeDtypeStruct((1, 4), jnp.float32),
    )(s, hp, p["Wr"], p["br"].reshape(1, -1), p["Wcls"], p["bcls"].reshape(1, -1))


_E = 640000
_CH = 128
_NCHT = _E // _CH
_NIT = 104
_RPT = N // 16


def _edge_sc_body(htab, ept_h, den_o, num_o,
                  ev0, ev1, ev2, rw0, rw1, rw2, mx0, mx1, mx2,
                  den_sh, num_sh, is0, is1, is2, gs0, gs1, gs2,
                  ss0, ss1, ss2):
    cid = lax.axis_index("c")
    sid = lax.axis_index("s")
    evs = (ev0, ev1, ev2)
    rws, mxs = (rw0, rw1, rw2), (mx0, mx1, mx2)
    isems, gsems, ssems = (is0, is1, is2), (gs0, gs1, gs2), (ss0, ss1, ss2)

    def iload(b, s_idx):
        off = jnp.minimum(s_idx, _NCHT - 1) * _CH
        pltpu.async_copy(ept_h.at[:, pl.ds(off, _CH)], evs[b], isems[b])

    def iload_wait(b, s_idx):
        off = jnp.minimum(s_idx, _NCHT - 1) * _CH
        pltpu.make_async_copy(ept_h.at[:, pl.ds(off, _CH)], evs[b],
                              isems[b]).wait()

    def gissue(b):
        pltpu.async_copy(htab.at[evs[b].at[cid]], rws[b], gsems[b])

    def gwait(b):
        pltpu.make_async_copy(htab.at[evs[b].at[cid]], rws[b], gsems[b]).wait()

    def compute(b):
        def crow(j2, _):
            for rr in range(4):
                r = 4 * j2 + rr
                for k in range(4):
                    v = rws[b][r, pl.ds(16 * k, 16)]
                    mm = jnp.maximum(v, 0.0) + GEN_EPS
                    e = jnp.exp(mm)
                    rws[b][r, pl.ds(16 * k, 16)] = e
                    mxs[b][r, pl.ds(16 * k, 16)] = mm * e
            return 0
        lax.fori_loop(0, _CH // 4, crow, 0)

    def sissue(b):
        pltpu.async_copy(rws[b], den_sh.at[evs[b].at[2]], ssems[b], add=True)
        pltpu.async_copy(mxs[b], num_sh.at[evs[b].at[2]], ssems[b], add=True)

    def swait(b):
        pltpu.make_async_copy(rws[b], den_sh.at[evs[b].at[2]], ssems[b]).wait()
        pltpu.make_async_copy(mxs[b], num_sh.at[evs[b].at[2]], ssems[b]).wait()

    def zb(j, _):
        for k in range(4):
            rw2[j, pl.ds(16 * k, 16)] = jnp.zeros((16,), jnp.float32)
            mx2[j, pl.ds(16 * k, 16)] = jnp.zeros((16,), jnp.float32)
        return 0
    lax.fori_loop(0, _CH, zb, 0)
    for row in range(3):
        for k in range(8):
            ev2[row, pl.ds(16 * k, 16)] = jnp.zeros((16,), jnp.int32)
    for m in range(_RPT // 125):
        r0 = sid * _RPT + m * 125
        pltpu.sync_copy(rw2.at[pl.ds(0, 125)], den_sh.at[pl.ds(r0, 125)])
        pltpu.sync_copy(rw2.at[pl.ds(0, 125)], num_sh.at[pl.ds(r0, 125)])
    plsc.subcore_barrier()

    sissue(2)
    iload(0, sid)
    iload(1, sid + 16)
    iload_wait(0, sid)
    gissue(0)
    iload_wait(1, sid + 16)
    gissue(1)

    def body(i, _):
        for b in (0, 1, 2):
            c = 3 * i + b
            pb = (b + 2) % 3
            s_next = sid + 16 * (c + 2)
            swait(pb)
            iload(pb, s_next)
            gwait(b)
            compute(b)
            sissue(b)
            iload_wait(pb, s_next)
            gissue(pb)
        return 0
    lax.fori_loop(0, _NIT, body, 0)

    swait(2)
    gwait(0)
    gwait(1)

    @pl.when(sid < 8)
    def _():
        compute(0)
        pltpu.sync_copy(rw0, den_sh.at[ev0.at[2]], add=True)
        pltpu.sync_copy(mx0, num_sh.at[ev0.at[2]], add=True)

    plsc.subcore_barrier()

    for m in range(_RPT // 125):
        r0 = sid * _RPT + m * 125
        pltpu.sync_copy(den_sh.at[pl.ds(r0, 125)], rw2.at[pl.ds(0, 125)])
        pltpu.sync_copy(rw2.at[pl.ds(0, 125)], den_o.at[cid, pl.ds(r0, 125)])
        pltpu.sync_copy(num_sh.at[pl.ds(r0, 125)], mx2.at[pl.ds(0, 125)])
        pltpu.sync_copy(mx2.at[pl.ds(0, 125)], num_o.at[cid, pl.ds(r0, 125)])


_edge_sc = pl.kernel(
    _edge_sc_body,
    out_type=[jax.ShapeDtypeStruct((2, N, 64), jnp.float32),
              jax.ShapeDtypeStruct((2, N, 64), jnp.float32)],
    mesh=plsc.VectorSubcoreMesh(core_axis_name="c", subcore_axis_name="s"),
    scratch_types=(
        [pltpu.VMEM((3, _CH), jnp.int32)] * 3
        + [pltpu.VMEM((_CH, 64), jnp.float32)] * 6
        + [pltpu.VMEM_SHARED((N, 64), jnp.float32)] * 2
        + [pltpu.SemaphoreType.DMA] * 9
    ),
    compiler_params=pltpu.CompilerParams(use_tc_tiling_on_sc=False),
)


def _edge_numden(h, ept):
    htab = h.reshape(2 * N, 64)
    den, num = _edge_sc(htab, ept)
    return den, num


def kernel(x, edge_index, params):
    p = params
    src, dst = edge_index[0], edge_index[1]
    ept = jnp.concatenate([(src * 2)[None], (src * 2 + 1)[None], dst[None]], 0)
    h0 = _fc(x, p["W_fc"], p["b_fc"])
    zeros_h = jnp.zeros((HID,), jnp.float32)
    ones_h = jnp.ones((HID,), jnp.float32)

    den, num = _edge_numden(h0, ept)
    h1 = _conv_dense(h0, den, num, p["conv0_W1"], p["conv0_b1"], p["conv0_g1"],
                     p["conv0_be1"], p["conv0_W2"], p["conv0_b2"],
                     ones_h, zeros_h, res=False)
    h = h1
    hs = [h0, h1]
    for i in (1, 2):
        den, num = _edge_numden(h, ept)
        h = _conv_dense(h, den, num, p["conv%d_W1" % i], p["conv%d_b1" % i],
                        p["conv%d_g1" % i], p["conv%d_be1" % i],
                        p["conv%d_W2" % i], p["conv%d_b2" % i],
                        p["norm%d_g" % i], p["norm%d_b" % i], res=True)
        hs.append(h)
    x_ = jnp.concatenate(hs, axis=1)
    hp, s = _head1(x_, p)
    return _head2(s, hp, p)


# speedup vs baseline: 1.0116x; 1.0116x over previous
import functools

import jax
import jax.numpy as jnp
from jax import lax
from jax.experimental import pallas as pl
from jax.experimental.pallas import tpu as pltpu
from jax.experimental.pallas import tpu_sc as plsc

N = 10000
HID = 128
D4 = 512
GEN_EPS = 1e-7
RB = 1000


def _ln(t, g, b, eps=1e-5):
    mu = jnp.mean(t, axis=-1, keepdims=True)
    var = jnp.mean((t - mu) ** 2, axis=-1, keepdims=True)
    return (t - mu) / jnp.sqrt(var + eps) * g + b


def _fc_body(x_ref, w_ref, b_ref, o_ref):
    o_ref[...] = jnp.maximum(
        jnp.dot(x_ref[...], w_ref[...], preferred_element_type=jnp.float32)
        + b_ref[...], 0.0)


def _fc(x, W, b):
    return pl.pallas_call(
        _fc_body,
        grid=(N // RB,),
        in_specs=[
            pl.BlockSpec((RB, 1024), lambda i: (i, 0)),
            pl.BlockSpec((1024, HID), lambda i: (0, 0)),
            pl.BlockSpec((1, HID), lambda i: (0, 0)),
        ],
        out_specs=pl.BlockSpec((RB, HID), lambda i: (i, 0)),
        out_shape=jax.ShapeDtypeStruct((N, HID), jnp.float32),
    )(x, W, b.reshape(1, HID))


def _conv_body(res, h_ref, den_ref, num_ref, w1_ref, b1_ref, g1_ref,
               be1_ref, w2_ref, b2_ref, ng_ref, nb_ref, o_ref):
    h = h_ref[...]
    den = jnp.concatenate([den_ref[0], den_ref[1]], axis=-1)
    num = jnp.concatenate([num_ref[0], num_ref[1]], axis=-1)
    agg = num / (den + 1e-16)
    out = agg + h
    t = jnp.dot(out, w1_ref[...], preferred_element_type=jnp.float32) + b1_ref[...]
    t = jnp.maximum(_ln(t, g1_ref[...], be1_ref[...]), 0.0)
    r = jnp.dot(t, w2_ref[...], preferred_element_type=jnp.float32) + b2_ref[...]
    if res:
        r = jnp.maximum(_ln(r, ng_ref[...], nb_ref[...]), 0.0)
        o_ref[...] = h + r
    else:
        o_ref[...] = r


def _conv_dense(h, den, num, w1, b1, g1, be1, w2, b2, ng, nb, res):
    return pl.pallas_call(
        functools.partial(_conv_body, res),
        grid=(N // RB,),
        in_specs=[
            pl.BlockSpec((RB, HID), lambda i: (i, 0)),
            pl.BlockSpec((2, RB, 64), lambda i: (0, i, 0)),
            pl.BlockSpec((2, RB, 64), lambda i: (0, i, 0)),
            pl.BlockSpec((HID, 2 * HID), lambda i: (0, 0)),
            pl.BlockSpec((1, 2 * HID), lambda i: (0, 0)),
            pl.BlockSpec((1, 2 * HID), lambda i: (0, 0)),
            pl.BlockSpec((1, 2 * HID), lambda i: (0, 0)),
            pl.BlockSpec((2 * HID, HID), lambda i: (0, 0)),
            pl.BlockSpec((1, HID), lambda i: (0, 0)),
            pl.BlockSpec((1, HID), lambda i: (0, 0)),
            pl.BlockSpec((1, HID), lambda i: (0, 0)),
        ],
        out_specs=pl.BlockSpec((RB, HID), lambda i: (i, 0)),
        out_shape=jax.ShapeDtypeStruct((N, HID), jnp.float32),
    )(h, den, num, w1, b1.reshape(1, -1), g1.reshape(1, -1),
      be1.reshape(1, -1), w2, b2.reshape(1, -1), ng.reshape(1, -1),
      nb.reshape(1, -1))


def _head1_body(x_ref, wphi_ref, bphi_ref, wa_ref, ba_ref, wb_ref, bb_ref,
                wc_ref, bc_ref, hp_ref, s_ref):
    hp = jnp.maximum(
        jnp.dot(x_ref[...], wphi_ref[...], preferred_element_type=jnp.float32)
        + bphi_ref[...], 0.0)
    a = jnp.tanh(jnp.dot(hp, wa_ref[...], preferred_element_type=jnp.float32)
                 + ba_ref[...])
    b = jax.nn.sigmoid(
        jnp.dot(hp, wb_ref[...], preferred_element_type=jnp.float32)
        + bb_ref[...])
    s = jnp.dot(a * b, wc_ref[...], preferred_element_type=jnp.float32) + bc_ref[...]
    hp_ref[...] = hp
    s_ref[...] = s


def _head1(x_, p):
    return pl.pallas_call(
        _head1_body,
        grid=(N // RB,),
        in_specs=[
            pl.BlockSpec((RB, D4), lambda i: (i, 0)),
            pl.BlockSpec((D4, D4), lambda i: (0, 0)),
            pl.BlockSpec((1, D4), lambda i: (0, 0)),
            pl.BlockSpec((D4, D4), lambda i: (0, 0)),
            pl.BlockSpec((1, D4), lambda i: (0, 0)),
            pl.BlockSpec((D4, D4), lambda i: (0, 0)),
            pl.BlockSpec((1, D4), lambda i: (0, 0)),
            pl.BlockSpec((D4, 1), lambda i: (0, 0)),
            pl.BlockSpec((1, 1), lambda i: (0, 0)),
        ],
        out_specs=[
            pl.BlockSpec((RB, D4), lambda i: (i, 0)),
            pl.BlockSpec((RB, 1), lambda i: (i, 0)),
        ],
        out_shape=[
            jax.ShapeDtypeStruct((N, D4), jnp.float32),
            jax.ShapeDtypeStruct((N, 1), jnp.float32),
        ],
    )(x_, p["W_phi"], p["b_phi"].reshape(1, -1), p["Wa"], p["ba"].reshape(1, -1),
      p["Wb"], p["bb"].reshape(1, -1), p["Wc"], p["bc"].reshape(1, -1))


def _head2_body(s_ref, hp_ref, wr_ref, br_ref, wcls_ref, bcls_ref, o_ref):
    s = s_ref[...]
    m = jnp.max(s)
    e = jnp.exp(s - m)
    w = e / jnp.sum(e)
    hpool = jnp.sum(w * hp_ref[...], axis=0, keepdims=True)
    hh = jnp.maximum(
        jnp.dot(hpool, wr_ref[...], preferred_element_type=jnp.float32)
        + br_ref[...], 0.0)
    o_ref[...] = (jnp.dot(hh, wcls_ref[...], preferred_element_type=jnp.float32)
                  + bcls_ref[...])


def _head2(s, hp, p):
    return pl.pallas_call(
        _head2_body,
        out_shape=jax.ShapeDtypeStruct((1, 4), jnp.float32),
    )(s, hp, p["Wr"], p["br"].reshape(1, -1), p["Wcls"], p["bcls"].reshape(1, -1))


_E = 640000
_CH = 128
_NCHT = _E // _CH
_NIT = 104
_RPT = N // 16


def _edge_sc_body(htab, src_h, dst_h, den_o, num_o,
                  sv0, sv1, sv2, dv0, dv1, dv2, ix0, ix1, ix2,
                  rw0, rw1, rw2, mx0, mx1, mx2,
                  den_sh, num_sh, is0, is1, is2, gs0, gs1, gs2,
                  ss0, ss1, ss2):
    cid = lax.axis_index("c")
    sid = lax.axis_index("s")
    svs, dvs, ixs = (sv0, sv1, sv2), (dv0, dv1, dv2), (ix0, ix1, ix2)
    rws, mxs = (rw0, rw1, rw2), (mx0, mx1, mx2)
    isems, gsems, ssems = (is0, is1, is2), (gs0, gs1, gs2), (ss0, ss1, ss2)

    def iload(b, s_idx):
        off = jnp.minimum(s_idx, _NCHT - 1) * _CH
        pltpu.async_copy(src_h.at[pl.ds(off, _CH)], svs[b], isems[b])
        pltpu.async_copy(dst_h.at[pl.ds(off, _CH)], dvs[b], isems[b])

    def iload_wait(b, s_idx):
        off = jnp.minimum(s_idx, _NCHT - 1) * _CH
        pltpu.make_async_copy(src_h.at[pl.ds(off, _CH)], svs[b], isems[b]).wait()
        pltpu.make_async_copy(dst_h.at[pl.ds(off, _CH)], dvs[b], isems[b]).wait()

    def transform(b):
        for k in range(8):
            sv = svs[b][pl.ds(16 * k, 16)]
            ixs[b][pl.ds(16 * k, 16)] = sv * 2 + cid

    def gissue(b):
        pltpu.async_copy(htab.at[ixs[b]], rws[b], gsems[b])

    def gwait(b):
        pltpu.make_async_copy(htab.at[ixs[b]], rws[b], gsems[b]).wait()

    def compute(b):
        def crow(j2, _):
            for rr in range(4):
                r = 4 * j2 + rr
                for k in range(4):
                    v = rws[b][r, pl.ds(16 * k, 16)]
                    mm = jnp.maximum(v, 0.0) + GEN_EPS
                    e = jnp.exp(mm)
                    rws[b][r, pl.ds(16 * k, 16)] = e
                    mxs[b][r, pl.ds(16 * k, 16)] = mm * e
            return 0
        lax.fori_loop(0, _CH // 4, crow, 0)

    def sissue(b):
        pltpu.async_copy(rws[b], den_sh.at[dvs[b]], ssems[b], add=True)
        pltpu.async_copy(mxs[b], num_sh.at[dvs[b]], ssems[b], add=True)

    def swait(b):
        pltpu.make_async_copy(rws[b], den_sh.at[dvs[b]], ssems[b]).wait()
        pltpu.make_async_copy(mxs[b], num_sh.at[dvs[b]], ssems[b]).wait()

    def zb(j, _):
        for k in range(4):
            rw2[j, pl.ds(16 * k, 16)] = jnp.zeros((16,), jnp.float32)
            mx2[j, pl.ds(16 * k, 16)] = jnp.zeros((16,), jnp.float32)
        return 0
    lax.fori_loop(0, _CH, zb, 0)
    for k in range(8):
        dv2[pl.ds(16 * k, 16)] = jnp.zeros((16,), jnp.int32)
    for m in range(_RPT // 125):
        r0 = sid * _RPT + m * 125
        pltpu.sync_copy(rw2.at[pl.ds(0, 125)], den_sh.at[pl.ds(r0, 125)])
        pltpu.sync_copy(rw2.at[pl.ds(0, 125)], num_sh.at[pl.ds(r0, 125)])
    plsc.subcore_barrier()

    sissue(2)
    iload(0, sid)
    iload(1, sid + 16)
    iload_wait(0, sid)
    transform(0)
    gissue(0)
    iload_wait(1, sid + 16)
    transform(1)
    gissue(1)

    def body(i, _):
        for b in (0, 1, 2):
            c = 3 * i + b
            pb = (b + 2) % 3
            s_next = sid + 16 * (c + 2)
            swait(pb)
            iload(pb, s_next)
            gwait(b)
            compute(b)
            sissue(b)
            iload_wait(pb, s_next)
            transform(pb)
            gissue(pb)
        return 0
    lax.fori_loop(0, _NIT, body, 0)

    swait(2)
    gwait(0)
    gwait(1)

    @pl.when(sid < 8)
    def _():
        compute(0)
        pltpu.sync_copy(rw0, den_sh.at[dv0], add=True)
        pltpu.sync_copy(mx0, num_sh.at[dv0], add=True)

    plsc.subcore_barrier()

    for m in range(_RPT // 125):
        r0 = sid * _RPT + m * 125
        pltpu.sync_copy(den_sh.at[pl.ds(r0, 125)], rw2.at[pl.ds(0, 125)])
        pltpu.sync_copy(rw2.at[pl.ds(0, 125)], den_o.at[cid, pl.ds(r0, 125)])
        pltpu.sync_copy(num_sh.at[pl.ds(r0, 125)], mx2.at[pl.ds(0, 125)])
        pltpu.sync_copy(mx2.at[pl.ds(0, 125)], num_o.at[cid, pl.ds(r0, 125)])


_edge_sc = pl.kernel(
    _edge_sc_body,
    out_type=[jax.ShapeDtypeStruct((2, N, 64), jnp.float32),
              jax.ShapeDtypeStruct((2, N, 64), jnp.float32)],
    mesh=plsc.VectorSubcoreMesh(core_axis_name="c", subcore_axis_name="s"),
    scratch_types=(
        [pltpu.VMEM((_CH,), jnp.int32)] * 6
        + [pltpu.VMEM((_CH,), jnp.int32)] * 3
        + [pltpu.VMEM((_CH, 64), jnp.float32)] * 6
        + [pltpu.VMEM_SHARED((N, 64), jnp.float32)] * 2
        + [pltpu.SemaphoreType.DMA] * 9
    ),
    compiler_params=pltpu.CompilerParams(use_tc_tiling_on_sc=False),
)


def _edge_numden(h, src, dst):
    htab = h.reshape(2 * N, 64)
    den, num = _edge_sc(htab, src, dst)
    return den, num


def kernel(x, edge_index, params):
    p = params
    src, dst = edge_index[0], edge_index[1]
    h0 = _fc(x, p["W_fc"], p["b_fc"])
    zeros_h = jnp.zeros((HID,), jnp.float32)
    ones_h = jnp.ones((HID,), jnp.float32)

    den, num = _edge_numden(h0, src, dst)
    h1 = _conv_dense(h0, den, num, p["conv0_W1"], p["conv0_b1"], p["conv0_g1"],
                     p["conv0_be1"], p["conv0_W2"], p["conv0_b2"],
                     ones_h, zeros_h, res=False)
    h = h1
    hs = [h0, h1]
    for i in (1, 2):
        den, num = _edge_numden(h, src, dst)
        h = _conv_dense(h, den, num, p["conv%d_W1" % i], p["conv%d_b1" % i],
                        p["conv%d_g1" % i], p["conv%d_be1" % i],
                        p["conv%d_W2" % i], p["conv%d_b2" % i],
                        p["norm%d_g" % i], p["norm%d_b" % i], res=True)
        hs.append(h)
    x_ = jnp.concatenate(hs, axis=1)
    hp, s = _head1(x_, p)
    return _head2(s, hp, p)
